# baseline (speedup 1.0000x reference)
import functools

import jax
import jax.numpy as jnp
from jax import lax
from jax.experimental import pallas as pl
from jax.experimental.pallas import tpu as pltpu
from jax.experimental.pallas import tpu_sc as plsc

_IN_DIM = 1000000
_OUT_DIM = 64
_BATCH = 16384

_NC = 2
_NS = 16
_NW = _NC * _NS
_L = 16

_LANE = 128
_NTILE = -(-_IN_DIM // _LANE)
_BASE_T = _NTILE // _NW
_EXTRA = _NTILE - _BASE_T * _NW
_PIECE_T = 1
_NBUF = 8
_PIECE_C = _PIECE_T * _LANE
_NPIECE = -(-(_BASE_T + 1) // _PIECE_T)
_MAX_START = (_IN_DIM - _PIECE_C) // _LANE
_TAIL_C = _NTILE * _LANE - _PIECE_C - _MAX_START * _LANE
_TAIL0 = _IN_DIM - (_IN_DIM % _LANE)
_RING = 32
_NIDX_V = _BATCH // _L
_NOCT = 7
_OCT_P = -(-_NPIECE // _NOCT)
_OCT_C = _OCT_P * _PIECE_C
_OCT_CAP = 448


def _extract(vec, lane):
    onehot = lax.iota(jnp.int32, _L) == lane
    return jnp.sum(jnp.where(onehot, vec, 0))


@functools.partial(
    pl.kernel,
    mesh=plsc.VectorSubcoreMesh(core_axis_name="c", subcore_axis_name="s"),
    out_type=jax.ShapeDtypeStruct((_BATCH * _OUT_DIM,), jnp.float32),
    scratch_types=[
        pltpu.VMEM((128, 128), jnp.int32),
        pltpu.VMEM((_BATCH + _L,), jnp.int32),
        pltpu.VMEM((_NBUF, _OUT_DIM, _PIECE_C), jnp.float32),
        pltpu.VMEM((_RING, _OUT_DIM), jnp.float32),
        pltpu.VMEM((_L,), jnp.int32),
        pltpu.VMEM((_NOCT * _OCT_CAP + _L,), jnp.int32),
        pltpu.VMEM((_OUT_DIM, _IN_DIM - _TAIL0), jnp.float32),
        pltpu.SemaphoreType.DMA((_NBUF,)),
        pltpu.SemaphoreType.DMA,
    ],
    compiler_params=pltpu.CompilerParams(use_tc_tiling_on_sc=True,
                                         needs_layout_passes=False),
)
def _sc_gather(idx_hbm, tablet_hbm, out_hbm, idx_v, hit_v, slab_v, stage_v,
               tmp_v, oct_v, tail_v, psem, osem):
    wid = lax.axis_index("s") * _NC + lax.axis_index("c")
    b0 = _BASE_T * wid + jnp.minimum(wid, _EXTRA)
    bt = _BASE_T + jnp.where(wid < _EXTRA, 1, 0)
    c_lo = b0 * _LANE
    c_hi = (b0 + bt) * _LANE
    iota = lax.iota(jnp.int32, _L)

    pltpu.sync_copy(idx_hbm, idx_v)

    def pre(k, m):
        vec = idx_v[lax.shift_right_logical(k, 3),
                    pl.ds(lax.bitwise_and(k, 7) * _L, _L)]
        inband = jnp.logical_and(vec >= c_lo, vec < c_hi)
        packed = lax.bitwise_or(lax.shift_left(vec - c_lo, 14), k * _L + iota)
        plsc.store_compressed(hit_v.at[pl.ds(m, _L)], packed, mask=inband)
        return m + jnp.sum(jnp.where(inband, 1, 0))

    m = lax.fori_loop(0, _NIDX_V, pre, 0)
    nvec = lax.shift_right_logical(m + _L - 1, 4)

    def pre2(j, st):
        ms, ovf = st
        hvec = hit_v[pl.ds(j * _L, _L)]
        valid = (j * _L + iota) < m
        lc = lax.shift_right_logical(hvec, 14)
        o = jnp.minimum(lax.div(lc, jnp.int32(_OCT_C)), _NOCT - 1)
        ms2 = []
        for ot in range(_NOCT):
            mo = ms[ot]
            msk = jnp.logical_and(o == ot, valid)
            cnt = jnp.sum(jnp.where(msk, 1, 0))
            room = mo <= _OCT_CAP - _L

            @pl.when(room)
            def _():
                plsc.store_compressed(oct_v.at[pl.ds(ot * _OCT_CAP + mo, _L)],
                                      hvec, mask=msk)

            ovf = jnp.logical_or(ovf,
                                 jnp.logical_and(jnp.logical_not(room),
                                                 cnt > 0))
            ms2.append(jnp.where(room, mo + cnt, mo))
        return tuple(ms2), ovf

    (moct, ovf) = lax.fori_loop(0, nvec, pre2,
                                ((jnp.int32(0),) * _NOCT, jnp.bool_(False)))
    nvec_full = jnp.where(ovf, nvec, 0)

    def start_col(p):
        t0 = b0 + jnp.minimum(_PIECE_T * p, bt - _PIECE_T)
        return jnp.minimum(t0, _MAX_START) * _LANE

    def fire(p):
        par = lax.rem(jnp.int32(p), _NBUF)
        pltpu.async_copy(tablet_hbm.at[:, pl.ds(start_col(p), _PIECE_C)],
                         slab_v.at[par], psem.at[par])

    def drain_out(n):
        def w(_, c):
            pltpu.make_async_copy(stage_v.at[0],
                                  out_hbm.at[pl.ds(0, _OUT_DIM)], osem).wait()
            return c

        lax.fori_loop(0, n, w, 0)

    def scan_piece(buf, p0, p1, state, listref, base, mcnt, nv):
        def scan(j, st):
            hvec = listref[pl.ds(base + j * _L, _L)]
            cols = lax.shift_right_logical(hvec, 14) + c_lo
            valid = (j * _L + iota) < mcnt
            inp = jnp.logical_and(jnp.logical_and(cols >= p0, cols < p1),
                                  valid)
            packed2 = lax.bitwise_or(lax.shift_left(cols - p0, 14),
                                     lax.bitwise_and(hvec, 16383))
            plsc.store_compressed(tmp_v.at[pl.ds(0, _L)], packed2, mask=inp)
            cnt = jnp.sum(jnp.where(inp, 1, 0))

            def hit(h, st2):
                outst, ring = st2
                hv = _extract(tmp_v[pl.ds(0, _L)], h)
                c = lax.shift_right_logical(hv, 14)
                pos = lax.bitwise_and(hv, 16383)
                csplat = jnp.full((_L,), c, jnp.int32)
                for q in range(_OUT_DIM // _L):
                    stage_v[ring, pl.ds(q * _L, _L)] = plsc.load_gather(
                        buf, [iota + q * _L, csplat])
                pltpu.async_copy(stage_v.at[ring],
                                 out_hbm.at[pl.ds(pos * _OUT_DIM, _OUT_DIM)],
                                 osem)
                outst = outst + 1
                wrap = ring + 1 == _RING

                @pl.when(wrap)
                def _():
                    drain_out(outst)

                return (jnp.where(wrap, 0, outst),
                        jnp.where(wrap, 0, ring + 1))

            return lax.fori_loop(0, cnt, hit, st)

        return lax.fori_loop(0, nv, scan, state)

    mvec = jnp.zeros((_L,), jnp.int32)
    for ot in range(_NOCT):
        mvec = mvec + jnp.where(iota == ot, moct[ot], 0)

    state = (jnp.int32(0), jnp.int32(0))
    for p in range(_NBUF - 1):
        fire(p)

    def piece_body(p, st):
        par = lax.rem(p, _NBUF)
        pltpu.make_async_copy(tablet_hbm.at[:, pl.ds(0, _PIECE_C)],
                              slab_v.at[par], psem.at[par]).wait()

        @pl.when(p + _NBUF - 1 < _NPIECE)
        def _():
            fire(p + _NBUF - 1)

        pc0 = start_col(p)
        o = lax.div(p, jnp.int32(_OCT_P))
        mo = _extract(mvec, o)
        st = scan_piece(slab_v.at[par], pc0, pc0 + _PIECE_C, st, oct_v,
                        o * _OCT_CAP, mo,
                        lax.shift_right_logical(mo + _L - 1, 4))
        st = scan_piece(slab_v.at[par], pc0, pc0 + _PIECE_C, st, hit_v, 0, m,
                        nvec_full)
        return st

    state = lax.fori_loop(0, _NPIECE, piece_body, state)

    @pl.when(c_hi > _TAIL0)
    def _():
        pltpu.sync_copy(tablet_hbm.at[:, pl.ds(_TAIL0, _IN_DIM - _TAIL0)],
                        tail_v)

    mo6 = moct[_NOCT - 1]
    state = scan_piece(tail_v, jnp.int32(_TAIL0), jnp.int32(_IN_DIM), state,
                       oct_v, (_NOCT - 1) * _OCT_CAP, mo6,
                       lax.shift_right_logical(mo6 + _L - 1, 4))
    state = scan_piece(tail_v, jnp.int32(_TAIL0), jnp.int32(_IN_DIM), state,
                       hit_v, 0, m, nvec_full)
    drain_out(state[0])


def kernel(x, ker):
    idx = jnp.reshape(x, (128, 128)).astype(jnp.int32)
    out = _sc_gather(idx, ker.T)
    return jnp.reshape(out, (_BATCH, _OUT_DIM))


# speedup vs baseline: 1.0454x; 1.0454x over previous
import functools

import jax
import jax.numpy as jnp
from jax import lax
from jax.experimental import pallas as pl
from jax.experimental.pallas import tpu as pltpu
from jax.experimental.pallas import tpu_sc as plsc

_IN_DIM = 1000000
_OUT_DIM = 64
_BATCH = 16384

_NC = 2
_NS = 16
_NW = _NC * _NS
_L = 16

_LANE = 128
_NTILE = -(-_IN_DIM // _LANE)
_BASE_T = _NTILE // _NW
_EXTRA = _NTILE - _BASE_T * _NW
_PIECE_T = 2
_NBUF = 6
_PIECE_C = _PIECE_T * _LANE
_NPIECE = -(-(_BASE_T + 1) // _PIECE_T)
_MAX_START = (_IN_DIM - _PIECE_C) // _LANE
_TAIL_C = _NTILE * _LANE - _PIECE_C - _MAX_START * _LANE
_TAIL0 = _IN_DIM - (_IN_DIM % _LANE)
_RING = 16
_NIDX_V = _BATCH // _L
_NOCT = 7
_OCT_P = -(-_NPIECE // _NOCT)
_OCT_C = _OCT_P * _PIECE_C
_OCT_CAP = 320


def _extract(vec, lane):
    onehot = lax.iota(jnp.int32, _L) == lane
    return jnp.sum(jnp.where(onehot, vec, 0))


@functools.partial(
    pl.kernel,
    mesh=plsc.VectorSubcoreMesh(core_axis_name="c", subcore_axis_name="s"),
    out_type=jax.ShapeDtypeStruct((_BATCH * _OUT_DIM,), jnp.float32),
    scratch_types=[
        pltpu.VMEM((2, 8, 128), jnp.int32),
        pltpu.VMEM((_BATCH + _L,), jnp.int32),
        pltpu.VMEM((_NBUF, _OUT_DIM, _PIECE_C), jnp.float32),
        pltpu.VMEM((_RING, _OUT_DIM), jnp.float32),
        pltpu.VMEM((_L,), jnp.int32),
        pltpu.VMEM((_NOCT * _OCT_CAP + _L,), jnp.int32),
        pltpu.VMEM((_OUT_DIM, _IN_DIM - _TAIL0), jnp.float32),
        pltpu.SemaphoreType.DMA((_NBUF,)),
        pltpu.SemaphoreType.DMA,
    ],
    compiler_params=pltpu.CompilerParams(use_tc_tiling_on_sc=True,
                                         needs_layout_passes=False),
)
def _sc_gather(idx_hbm, tablet_hbm, out_hbm, idx_v, hit_v, slab_v, stage_v,
               tmp_v, oct_v, tail_v, psem, osem):
    wid = lax.axis_index("s") * _NC + lax.axis_index("c")
    b0 = _BASE_T * wid + jnp.minimum(wid, _EXTRA)
    bt = _BASE_T + jnp.where(wid < _EXTRA, 1, 0)
    c_lo = b0 * _LANE
    c_hi = (b0 + bt) * _LANE
    iota = lax.iota(jnp.int32, _L)

    pltpu.async_copy(idx_hbm.at[pl.ds(0, 8)], idx_v.at[0], psem.at[0])
    m = jnp.int32(0)
    for ck in range(16):
        pltpu.make_async_copy(idx_hbm.at[pl.ds(0, 8)], idx_v.at[ck % 2],
                              psem.at[ck % 2]).wait()
        if ck + 1 < 16:
            pltpu.async_copy(idx_hbm.at[pl.ds(8 * (ck + 1), 8)],
                             idx_v.at[(ck + 1) % 2], psem.at[(ck + 1) % 2])

        def pre(j, mm, ck=ck):
            vec = idx_v[ck % 2, lax.shift_right_logical(j, 3),
                        pl.ds(lax.bitwise_and(j, 7) * _L, _L)]
            inband = jnp.logical_and(vec >= c_lo, vec < c_hi)
            packed = lax.bitwise_or(lax.shift_left(vec - c_lo, 14),
                                    (ck * 64 + j) * _L + iota)
            plsc.store_compressed(hit_v.at[pl.ds(mm, _L)], packed,
                                  mask=inband)
            return mm + jnp.sum(jnp.where(inband, 1, 0))

        m = lax.fori_loop(0, 64, pre, m)
    nvec = lax.shift_right_logical(m + _L - 1, 4)

    def pre2(j, st):
        ms, ovf = st
        hvec = hit_v[pl.ds(j * _L, _L)]
        valid = (j * _L + iota) < m
        lc = lax.shift_right_logical(hvec, 14)
        o = jnp.minimum(lax.div(lc, jnp.int32(_OCT_C)), _NOCT - 1)
        ms2 = []
        for ot in range(_NOCT):
            mo = ms[ot]
            msk = jnp.logical_and(o == ot, valid)
            cnt = jnp.sum(jnp.where(msk, 1, 0))
            room = mo <= _OCT_CAP - _L

            @pl.when(room)
            def _():
                plsc.store_compressed(oct_v.at[pl.ds(ot * _OCT_CAP + mo, _L)],
                                      hvec, mask=msk)

            ovf = jnp.logical_or(ovf,
                                 jnp.logical_and(jnp.logical_not(room),
                                                 cnt > 0))
            ms2.append(jnp.where(room, mo + cnt, mo))
        return tuple(ms2), ovf

    (moct, ovf) = lax.fori_loop(0, nvec, pre2,
                                ((jnp.int32(0),) * _NOCT, jnp.bool_(False)))
    nvec_full = jnp.where(ovf, nvec, 0)

    def start_col(p):
        t0 = b0 + jnp.minimum(_PIECE_T * p, bt - _PIECE_T)
        return jnp.minimum(t0, _MAX_START) * _LANE

    def fire(p):
        par = lax.rem(jnp.int32(p), _NBUF)
        pltpu.async_copy(tablet_hbm.at[:, pl.ds(start_col(p), _PIECE_C)],
                         slab_v.at[par], psem.at[par])

    def drain_out(n):
        def w(_, c):
            pltpu.make_async_copy(stage_v.at[0],
                                  out_hbm.at[pl.ds(0, _OUT_DIM)], osem).wait()
            return c

        lax.fori_loop(0, n, w, 0)

    def scan_piece(buf, p0, p1, state, listref, base, mcnt, nv):
        def scan(j, st):
            hvec = listref[pl.ds(base + j * _L, _L)]
            cols = lax.shift_right_logical(hvec, 14) + c_lo
            valid = (j * _L + iota) < mcnt
            inp = jnp.logical_and(jnp.logical_and(cols >= p0, cols < p1),
                                  valid)
            packed2 = lax.bitwise_or(lax.shift_left(cols - p0, 14),
                                     lax.bitwise_and(hvec, 16383))
            plsc.store_compressed(tmp_v.at[pl.ds(0, _L)], packed2, mask=inp)
            cnt = jnp.sum(jnp.where(inp, 1, 0))

            def hit(h, st2):
                outst, ring = st2
                hv = _extract(tmp_v[pl.ds(0, _L)], h)
                c = lax.shift_right_logical(hv, 14)
                pos = lax.bitwise_and(hv, 16383)
                csplat = jnp.full((_L,), c, jnp.int32)
                for q in range(_OUT_DIM // _L):
                    stage_v[ring, pl.ds(q * _L, _L)] = plsc.load_gather(
                        buf, [iota + q * _L, csplat])
                pltpu.async_copy(stage_v.at[ring],
                                 out_hbm.at[pl.ds(pos * _OUT_DIM, _OUT_DIM)],
                                 osem)
                outst = outst + 1
                wrap = ring + 1 == _RING

                @pl.when(wrap)
                def _():
                    drain_out(outst)

                return (jnp.where(wrap, 0, outst),
                        jnp.where(wrap, 0, ring + 1))

            return lax.fori_loop(0, cnt, hit, st)

        return lax.fori_loop(0, nv, scan, state)

    mvec = jnp.zeros((_L,), jnp.int32)
    for ot in range(_NOCT):
        mvec = mvec + jnp.where(iota == ot, moct[ot], 0)

    state = (jnp.int32(0), jnp.int32(0))
    for p in range(_NBUF - 1):
        fire(p)

    def piece_body(p, st):
        par = lax.rem(p, _NBUF)
        pltpu.make_async_copy(tablet_hbm.at[:, pl.ds(0, _PIECE_C)],
                              slab_v.at[par], psem.at[par]).wait()

        @pl.when(p + _NBUF - 1 < _NPIECE)
        def _():
            fire(p + _NBUF - 1)

        pc0 = start_col(p)
        o = lax.div(p, jnp.int32(_OCT_P))
        mo = _extract(mvec, o)
        st = scan_piece(slab_v.at[par], pc0, pc0 + _PIECE_C, st, oct_v,
                        o * _OCT_CAP, mo,
                        lax.shift_right_logical(mo + _L - 1, 4))
        st = scan_piece(slab_v.at[par], pc0, pc0 + _PIECE_C, st, hit_v, 0, m,
                        nvec_full)
        return st

    state = lax.fori_loop(0, _NPIECE, piece_body, state)

    @pl.when(c_hi > _TAIL0)
    def _():
        pltpu.sync_copy(tablet_hbm.at[:, pl.ds(_TAIL0, _IN_DIM - _TAIL0)],
                        tail_v)

    mo6 = moct[_NOCT - 1]
    state = scan_piece(tail_v, jnp.int32(_TAIL0), jnp.int32(_IN_DIM), state,
                       oct_v, (_NOCT - 1) * _OCT_CAP, mo6,
                       lax.shift_right_logical(mo6 + _L - 1, 4))
    state = scan_piece(tail_v, jnp.int32(_TAIL0), jnp.int32(_IN_DIM), state,
                       hit_v, 0, m, nvec_full)
    drain_out(state[0])


def kernel(x, ker):
    idx = jnp.reshape(x, (128, 128)).astype(jnp.int32)
    out = _sc_gather(idx, ker.T)
    return jnp.reshape(out, (_BATCH, _OUT_DIM))
